# baseline (speedup 1.0000x reference)
import jax
import jax.numpy as jnp
from jax.experimental import pallas as pl
from jax.experimental.pallas import tpu as pltpu

N = 8192
E = 8192
BK = 256
NSTEPS = N // BK
NPH = 5
TOT = NPH * NSTEPS
NBUF = 3
_EPS = 1e-5


def _dot(a, b):
    return jax.lax.dot_general(
        a, b, (((1,), (0,)), ((), ())),
        precision=jax.lax.Precision.DEFAULT,
        preferred_element_type=jnp.float32)


def _bn_relu_max(zc, g, be):
    m = jnp.mean(zc, axis=0, keepdims=True)
    v = jnp.mean(jnp.square(zc), axis=0, keepdims=True) - jnp.square(m)
    zp = jax.nn.relu((zc - m) * jax.lax.rsqrt(v + _EPS) * g + be)
    return jnp.max(zp, axis=1, keepdims=True)


def _prep_body(xn_ref, xe_ref, w1t_ref, hw1t_ref, hb1_ref,
               w2t_ref, b1v_ref, b2v_ref, g1_ref, be1_ref,
               hw2t_ref, hb2_ref, g2_ref, be2_ref,
               ehwt_ref, ehb_ref, nhwt_ref, nhb_ref,
               panel_ref, wp_ref):
    panel_ref[:, :32] = _dot(xn_ref[:], w1t_ref[:])
    panel_ref[:, 32:40] = _dot(xe_ref[:], hw1t_ref[:]) + hb1_ref[:]
    panel_ref[:, 40:41] = jnp.ones((E, 1), jnp.float32)
    panel_ref[:, 41:48] = jnp.zeros((E, 7), jnp.float32)
    wp_ref[0:32, 0:32] = w2t_ref[:]
    wp_ref[32:33, 0:32] = b1v_ref[:]
    wp_ref[33:34, 0:32] = b2v_ref[:]
    wp_ref[34:35, 0:8] = g1_ref[:]
    wp_ref[35:36, 0:8] = be1_ref[:]
    wp_ref[36:37, 0:8] = hw2t_ref[:]
    wp_ref[37:38, 0:8] = hb2_ref[:]
    wp_ref[38:39, 0:8] = g2_ref[:]
    wp_ref[39:40, 0:8] = be2_ref[:]
    wp_ref[0:34, 32:33] = nhwt_ref[:]
    wp_ref[34:36, 32:33] = ehwt_ref[:]
    wp_ref[36:37, 32:33] = ehb_ref[:]
    wp_ref[37:38, 32:33] = nhb_ref[:]


def _slab(hbm_ref, bufs, sems, slot, col):
    return pltpu.make_async_copy(
        hbm_ref.at[:, pl.ds(col * BK, BK)],
        bufs.at[slot],
        sems.at[slot])


def _issue(a_hbm, l1_hbm, b1_hbm, bufs, sems, s):
    ps = s // NSTEPS
    kk = s % NSTEPS
    slot = s % NBUF

    @pl.when((s < TOT) & (ps <= 1))
    def _():
        _slab(a_hbm, bufs, sems, slot, kk).start()

    @pl.when((s < TOT) & ((ps == 2) | (ps == 3)))
    def _():
        _slab(l1_hbm, bufs, sems, slot, kk).start()

    @pl.when((s < TOT) & (ps == 4))
    def _():
        _slab(b1_hbm, bufs, sems, slot, kk).start()


def _mega_body(panel0_ref, wp_ref,
               a_hbm, l1_hbm, b1m_hbm,
               out_ref,
               acc, yw, oz, bufs, sems):
    p = pl.program_id(0)
    k = pl.program_id(1)
    s = p * NSTEPS + k
    rows = pl.ds(k * BK, BK)

    @pl.when(s == 0)
    def _():
        for d in range(NBUF - 1):
            _issue(a_hbm, l1_hbm, b1m_hbm, bufs, sems, d)

    _issue(a_hbm, l1_hbm, b1m_hbm, bufs, sems, s + NBUF - 1)

    @pl.when(k == 0)
    def _():
        acc[:, :] = jnp.zeros((N, 32), jnp.float32)

    _slab(a_hbm, bufs, sems, s % NBUF, k).wait()
    blk = bufs[s % NBUF]

    @pl.when(p == 0)
    def _():
        acc[:, 0:32] = acc[:, 0:32] + _dot(blk, panel0_ref[rows, 0:32])

    @pl.when(p == 1)
    def _():
        acc[:, 0:32] = acc[:, 0:32] + _dot(blk, yw[rows, :])

    @pl.when(p == 2)
    def _():
        acc[:, 0:9] = acc[:, 0:9] + _dot(blk, panel0_ref[rows, 32:41])

    @pl.when(p == 3)
    def _():
        acc[:, 0:1] = acc[:, 0:1] + _dot(blk, oz[rows, 0:1])

    @pl.when(p == 4)
    def _():
        acc[:, 0:2] = acc[:, 0:2] + _dot(blk, oz[rows, 0:2])

    @pl.when((p == 0) & (k == NSTEPS - 1))
    def _():
        y1 = jax.nn.relu(acc[:, 0:32] + wp_ref[32:33, 0:32])
        yw[:, :] = _dot(y1, wp_ref[0:32, 0:32])

    @pl.when((p == 1) & (k == NSTEPS - 1))
    def _():
        out_ref[:, :32] = jax.nn.relu(acc[:, 0:32] + wp_ref[33:34, 0:32])

    @pl.when((p == 2) & (k == NSTEPS - 1))
    def _():
        oz[:, 0:1] = _bn_relu_max(acc[:, 0:8], wp_ref[34:35, 0:8],
                                  wp_ref[35:36, 0:8])
        oz[:, 2:3] = acc[:, 8:9]

    @pl.when((p == 3) & (k == NSTEPS - 1))
    def _():
        zc2 = (acc[:, 0:1] * wp_ref[36:37, 0:8]
               + oz[:, 2:3] * wp_ref[37:38, 0:8])
        oz[:, 1:2] = _bn_relu_max(zc2, wp_ref[38:39, 0:8],
                                  wp_ref[39:40, 0:8])
        out_ref[:, 35:36] = jax.nn.sigmoid(
            _dot(oz[:, 0:2], wp_ref[34:36, 32:33]) + wp_ref[36:37, 32:33])

    @pl.when((p == 4) & (k == NSTEPS - 1))
    def _():
        out_ref[:, 32:34] = acc[:, 0:2]
        out_ref[:, 34:35] = jax.nn.sigmoid(
            _dot(out_ref[:, 0:34], wp_ref[0:34, 32:33])
            + wp_ref[37:38, 32:33])
        out_ref[:, 36:40] = jnp.zeros((N, 4), jnp.float32)


def _full(shape):
    return pl.BlockSpec(shape, lambda *_: (0,) * len(shape))


_HBM = pl.BlockSpec(memory_space=pl.ANY)


def kernel(X_n, X_e, A_tilde, L1_tilde, B1, gnn_w1, gnn_b1, gnn_w2, gnn_b2,
           hosc1_w, hosc1_b, hosc1_g, hosc1_be, hosc2_w, hosc2_b, hosc2_g,
           hosc2_be, nh_w, nh_b, eh_w, eh_b):
    f32 = jnp.float32

    panel0, wpack = pl.pallas_call(
        _prep_body,
        in_specs=[_full((N, 128)), _full((E, 16)), _full((128, 32)),
                  _full((16, 8)), _full((1, 8)),
                  _full((32, 32)), _full((1, 32)), _full((1, 32)),
                  _full((1, 8)), _full((1, 8)), _full((1, 8)),
                  _full((1, 8)), _full((1, 8)), _full((1, 8)),
                  _full((2, 1)), _full((1, 1)), _full((34, 1)),
                  _full((1, 1))],
        out_specs=[_full((N, 48)), _full((40, 64))],
        out_shape=[jax.ShapeDtypeStruct((N, 48), f32),
                   jax.ShapeDtypeStruct((40, 64), f32)],
    )(X_n, X_e, gnn_w1.T, hosc1_w.T, hosc1_b.reshape(1, -1),
      gnn_w2.T, gnn_b1.reshape(1, -1), gnn_b2.reshape(1, -1),
      hosc1_g.reshape(1, -1), hosc1_be.reshape(1, -1), hosc2_w.T,
      hosc2_b.reshape(1, -1), hosc2_g.reshape(1, -1),
      hosc2_be.reshape(1, -1), eh_w.T, eh_b.reshape(1, -1), nh_w.T,
      nh_b.reshape(1, -1))

    out = pl.pallas_call(
        _mega_body,
        grid=(NPH, NSTEPS),
        in_specs=[_full((N, 48)), _full((40, 64)),
                  _HBM, _HBM, _HBM],
        out_specs=_full((N, 40)),
        out_shape=jax.ShapeDtypeStruct((N, 40), f32),
        scratch_shapes=[pltpu.VMEM((N, 32), f32),
                        pltpu.VMEM((N, 32), f32),
                        pltpu.VMEM((N, 8), f32),
                        pltpu.VMEM((NBUF, N, BK), f32),
                        pltpu.SemaphoreType.DMA((NBUF,))],
    )(panel0, wpack, A_tilde, L1_tilde, B1)

    return out[:, 34], out[:, 35], out[:, :34]


# speedup vs baseline: 1.0385x; 1.0385x over previous
import jax
import jax.numpy as jnp
from jax.experimental import pallas as pl
from jax.experimental.pallas import tpu as pltpu

N = 8192
E = 8192
BK = 256
NSTEPS = N // BK
NPH = 5
TOT = NPH * NSTEPS
NBUF = 3
_EPS = 1e-5


def _dot(a, b):
    return jax.lax.dot_general(
        a, b, (((1,), (0,)), ((), ())),
        precision=jax.lax.Precision.DEFAULT,
        preferred_element_type=jnp.float32)


def _bn_relu_max(zc, g, be):
    m = jnp.mean(zc, axis=0, keepdims=True)
    v = jnp.mean(jnp.square(zc), axis=0, keepdims=True) - jnp.square(m)
    zp = jax.nn.relu((zc - m) * jax.lax.rsqrt(v + _EPS) * g + be)
    return jnp.max(zp, axis=1, keepdims=True)


def _slab(hbm_ref, bufs, sems, slot, col):
    return pltpu.make_async_copy(
        hbm_ref.at[:, pl.ds(col * BK, BK)],
        bufs.at[slot],
        sems.at[slot])


def _issue(a_hbm, l1_hbm, b1_hbm, bufs, sems, s):
    ps = s // NSTEPS
    kk = s % NSTEPS
    slot = s % NBUF

    @pl.when((s < TOT) & (ps <= 1))
    def _():
        _slab(a_hbm, bufs, sems, slot, kk).start()

    @pl.when((s < TOT) & ((ps == 2) | (ps == 3)))
    def _():
        _slab(l1_hbm, bufs, sems, slot, kk).start()

    @pl.when((s < TOT) & (ps == 4))
    def _():
        _slab(b1_hbm, bufs, sems, slot, kk).start()


def _mega_body(w1t_ref, b1v_ref, w2t_ref, b2v_ref, hw1t_ref, hb1_ref,
               g1_ref, be1_ref, hw2t_ref, hb2_ref, g2_ref, be2_ref,
               ehwt_ref, ehb_ref, nhwt_ref, nhb_ref,
               xn_hbm, xe_hbm, a_hbm, l1_hbm, b1m_hbm,
               out_ref,
               opnd, acc, oz, xbuf, bufs, sems, xsems):
    p = pl.program_id(0)
    k = pl.program_id(1)
    s = p * NSTEPS + k
    rows = pl.ds(k * BK, BK)

    @pl.when(s == 0)
    def _():
        cx = pltpu.make_async_copy(xn_hbm, xbuf.at[:, pl.ds(0, 128)],
                                   xsems.at[0])
        ce = pltpu.make_async_copy(xe_hbm, xbuf.at[:, pl.ds(128, 16)],
                                   xsems.at[1])
        cx.start()
        ce.start()
        for d in range(NBUF - 1):
            _issue(a_hbm, l1_hbm, b1m_hbm, bufs, sems, d)
        cx.wait()
        opnd[:, :] = _dot(xbuf[:, 0:128], w1t_ref[:])

    _issue(a_hbm, l1_hbm, b1m_hbm, bufs, sems, s + NBUF - 1)

    @pl.when(k == 0)
    def _():
        acc[:, :] = jnp.zeros((N, 32), jnp.float32)

    _slab(a_hbm, bufs, sems, s % NBUF, k).wait()
    acc[:, :] = acc[:, :] + _dot(bufs[s % NBUF], opnd[rows, :])

    @pl.when((p == 0) & (k == NSTEPS - 1))
    def _():
        y1 = jax.nn.relu(acc[:, :] + b1v_ref[:])
        opnd[:, :] = _dot(y1, w2t_ref[:])

    @pl.when((p == 1) & (k == NSTEPS - 1))
    def _():
        out_ref[:, :32] = jax.nn.relu(acc[:, :] + b2v_ref[:])
        pltpu.make_async_copy(xe_hbm, xbuf.at[:, pl.ds(128, 16)],
                              xsems.at[1]).wait()
        opnd[:, 0:8] = _dot(xbuf[:, 128:144], hw1t_ref[:]) + hb1_ref[:]
        opnd[:, 8:9] = jnp.ones((E, 1), jnp.float32)
        opnd[:, 9:32] = jnp.zeros((E, 23), jnp.float32)

    @pl.when((p == 2) & (k == NSTEPS - 1))
    def _():
        oz[:, 0:1] = acc[:, 8:9]
        opnd[:, 0:1] = _bn_relu_max(acc[:, 0:8], g1_ref[:], be1_ref[:])
        opnd[:, 1:32] = jnp.zeros((E, 31), jnp.float32)

    @pl.when((p == 3) & (k == NSTEPS - 1))
    def _():
        zc2 = acc[:, 0:1] * hw2t_ref[:] + oz[:, 0:1] * hb2_ref[:]
        opnd[:, 1:2] = _bn_relu_max(zc2, g2_ref[:], be2_ref[:])
        out_ref[:, 35:36] = jax.nn.sigmoid(
            _dot(opnd[:, 0:2], ehwt_ref[:]) + ehb_ref[:])

    @pl.when((p == 4) & (k == NSTEPS - 1))
    def _():
        out_ref[:, 32:34] = acc[:, 0:2]
        out_ref[:, 34:35] = jax.nn.sigmoid(
            _dot(out_ref[:, 0:34], nhwt_ref[:]) + nhb_ref[:])
        out_ref[:, 36:40] = jnp.zeros((N, 4), jnp.float32)


def _full(shape):
    return pl.BlockSpec(shape, lambda *_: (0,) * len(shape))


_HBM = pl.BlockSpec(memory_space=pl.ANY)


def kernel(X_n, X_e, A_tilde, L1_tilde, B1, gnn_w1, gnn_b1, gnn_w2, gnn_b2,
           hosc1_w, hosc1_b, hosc1_g, hosc1_be, hosc2_w, hosc2_b, hosc2_g,
           hosc2_be, nh_w, nh_b, eh_w, eh_b):
    f32 = jnp.float32

    out = pl.pallas_call(
        _mega_body,
        grid=(NPH, NSTEPS),
        in_specs=[_full((128, 32)), _full((1, 32)), _full((32, 32)),
                  _full((1, 32)), _full((16, 8)), _full((1, 8)),
                  _full((1, 8)), _full((1, 8)), _full((1, 8)),
                  _full((1, 8)), _full((1, 8)), _full((1, 8)),
                  _full((2, 1)), _full((1, 1)), _full((34, 1)),
                  _full((1, 1)),
                  _HBM, _HBM, _HBM, _HBM, _HBM],
        out_specs=_full((N, 40)),
        out_shape=jax.ShapeDtypeStruct((N, 40), f32),
        scratch_shapes=[pltpu.VMEM((N, 32), f32),
                        pltpu.VMEM((N, 32), f32),
                        pltpu.VMEM((N, 8), f32),
                        pltpu.VMEM((N, 144), f32),
                        pltpu.VMEM((NBUF, N, BK), f32),
                        pltpu.SemaphoreType.DMA((NBUF,)),
                        pltpu.SemaphoreType.DMA((2,))],
    )(gnn_w1.T, gnn_b1.reshape(1, -1), gnn_w2.T, gnn_b2.reshape(1, -1),
      hosc1_w.T, hosc1_b.reshape(1, -1), hosc1_g.reshape(1, -1),
      hosc1_be.reshape(1, -1), hosc2_w.T, hosc2_b.reshape(1, -1),
      hosc2_g.reshape(1, -1), hosc2_be.reshape(1, -1), eh_w.T,
      eh_b.reshape(1, -1), nh_w.T, nh_b.reshape(1, -1),
      X_n, X_e, A_tilde, L1_tilde, B1)

    return out[:, 34], out[:, 35], out[:, :34]


# speedup vs baseline: 1.0405x; 1.0019x over previous
import jax
import jax.numpy as jnp
from jax.experimental import pallas as pl
from jax.experimental.pallas import tpu as pltpu

N = 8192
E = 8192
BK = 256
NSTEPS = N // BK
NPH = 5
TOT = NPH * NSTEPS
NBUF = 4
_EPS = 1e-5


def _dot(a, b):
    return jax.lax.dot_general(
        a, b, (((1,), (0,)), ((), ())),
        precision=jax.lax.Precision.DEFAULT,
        preferred_element_type=jnp.float32)


def _bn_relu_max(zc, g, be):
    m = jnp.mean(zc, axis=0, keepdims=True)
    v = jnp.mean(jnp.square(zc), axis=0, keepdims=True) - jnp.square(m)
    zp = jax.nn.relu((zc - m) * jax.lax.rsqrt(v + _EPS) * g + be)
    return jnp.max(zp, axis=1, keepdims=True)


def _slab(hbm_ref, bufs, sems, slot, col):
    return pltpu.make_async_copy(
        hbm_ref.at[:, pl.ds(col * BK, BK)],
        bufs.at[slot],
        sems.at[slot])


def _issue(a_hbm, l1_hbm, b1_hbm, bufs, sems, s):
    ps = s // NSTEPS
    kk = s % NSTEPS
    slot = s % NBUF

    @pl.when((s < TOT) & (ps <= 1))
    def _():
        _slab(a_hbm, bufs, sems, slot, kk).start()

    @pl.when((s < TOT) & ((ps == 2) | (ps == 3)))
    def _():
        _slab(l1_hbm, bufs, sems, slot, kk).start()

    @pl.when((s < TOT) & (ps == 4))
    def _():
        _slab(b1_hbm, bufs, sems, slot, kk).start()


def _mega_body(w1t_ref, b1v_ref, w2t_ref, b2v_ref, hw1t_ref, hb1_ref,
               g1_ref, be1_ref, hw2t_ref, hb2_ref, g2_ref, be2_ref,
               ehwt_ref, ehb_ref, nhwt_ref, nhb_ref,
               xn_hbm, xe_hbm, a_hbm, l1_hbm, b1m_hbm,
               out_ref,
               opnd, acc, oz, xbuf, bufs, sems, xsems):
    p = pl.program_id(0)
    k = pl.program_id(1)
    s = p * NSTEPS + k
    rows = pl.ds(k * BK, BK)

    @pl.when(s == 0)
    def _():
        cx = pltpu.make_async_copy(xn_hbm, xbuf.at[:, pl.ds(0, 128)],
                                   xsems.at[0])
        ce = pltpu.make_async_copy(xe_hbm, oz.at[:, pl.ds(0, 16)],
                                   xsems.at[1])
        cx.start()
        ce.start()
        for d in range(NBUF - 1):
            _issue(a_hbm, l1_hbm, b1m_hbm, bufs, sems, d)
        cx.wait()
        opnd[:, :] = _dot(xbuf[:, :], w1t_ref[:])

    _issue(a_hbm, l1_hbm, b1m_hbm, bufs, sems, s + NBUF - 1)

    @pl.when(k == 0)
    def _():
        acc[:, :] = jnp.zeros((N, 32), jnp.float32)

    _slab(a_hbm, bufs, sems, s % NBUF, k).wait()
    acc[:, :] = acc[:, :] + _dot(bufs[s % NBUF], opnd[rows, :])

    @pl.when((p == 0) & (k == NSTEPS - 1))
    def _():
        y1 = jax.nn.relu(acc[:, :] + b1v_ref[:])
        opnd[:, :] = _dot(y1, w2t_ref[:])

    @pl.when((p == 1) & (k == NSTEPS - 1))
    def _():
        out_ref[:, :32] = jax.nn.relu(acc[:, :] + b2v_ref[:])
        pltpu.make_async_copy(xe_hbm, oz.at[:, pl.ds(0, 16)],
                              xsems.at[1]).wait()
        opnd[:, 0:8] = _dot(oz[:, 0:16], hw1t_ref[:]) + hb1_ref[:]
        opnd[:, 8:9] = jnp.ones((E, 1), jnp.float32)
        opnd[:, 9:32] = jnp.zeros((E, 23), jnp.float32)

    @pl.when((p == 2) & (k == NSTEPS - 1))
    def _():
        oz[:, 0:1] = acc[:, 8:9]
        opnd[:, 0:1] = _bn_relu_max(acc[:, 0:8], g1_ref[:], be1_ref[:])
        opnd[:, 1:32] = jnp.zeros((E, 31), jnp.float32)

    @pl.when((p == 3) & (k == NSTEPS - 1))
    def _():
        zc2 = acc[:, 0:1] * hw2t_ref[:] + oz[:, 0:1] * hb2_ref[:]
        opnd[:, 1:2] = _bn_relu_max(zc2, g2_ref[:], be2_ref[:])
        out_ref[:, 35:36] = jax.nn.sigmoid(
            _dot(opnd[:, 0:2], ehwt_ref[:]) + ehb_ref[:])

    @pl.when((p == 4) & (k == NSTEPS - 1))
    def _():
        out_ref[:, 32:34] = acc[:, 0:2]
        out_ref[:, 34:35] = jax.nn.sigmoid(
            _dot(out_ref[:, 0:34], nhwt_ref[:]) + nhb_ref[:])
        out_ref[:, 36:40] = jnp.zeros((N, 4), jnp.float32)


def _full(shape):
    return pl.BlockSpec(shape, lambda *_: (0,) * len(shape))


_HBM = pl.BlockSpec(memory_space=pl.ANY)


def kernel(X_n, X_e, A_tilde, L1_tilde, B1, gnn_w1, gnn_b1, gnn_w2, gnn_b2,
           hosc1_w, hosc1_b, hosc1_g, hosc1_be, hosc2_w, hosc2_b, hosc2_g,
           hosc2_be, nh_w, nh_b, eh_w, eh_b):
    f32 = jnp.float32

    out = pl.pallas_call(
        _mega_body,
        grid=(NPH, NSTEPS),
        in_specs=[_full((128, 32)), _full((1, 32)), _full((32, 32)),
                  _full((1, 32)), _full((16, 8)), _full((1, 8)),
                  _full((1, 8)), _full((1, 8)), _full((1, 8)),
                  _full((1, 8)), _full((1, 8)), _full((1, 8)),
                  _full((2, 1)), _full((1, 1)), _full((34, 1)),
                  _full((1, 1)),
                  _HBM, _HBM, _HBM, _HBM, _HBM],
        out_specs=_full((N, 40)),
        out_shape=jax.ShapeDtypeStruct((N, 40), f32),
        scratch_shapes=[pltpu.VMEM((N, 32), f32),
                        pltpu.VMEM((N, 32), f32),
                        pltpu.VMEM((N, 16), f32),
                        pltpu.VMEM((N, 128), f32),
                        pltpu.VMEM((NBUF, N, BK), f32),
                        pltpu.SemaphoreType.DMA((NBUF,)),
                        pltpu.SemaphoreType.DMA((2,))],
    )(gnn_w1.T, gnn_b1.reshape(1, -1), gnn_w2.T, gnn_b2.reshape(1, -1),
      hosc1_w.T, hosc1_b.reshape(1, -1), hosc1_g.reshape(1, -1),
      hosc1_be.reshape(1, -1), hosc2_w.T, hosc2_b.reshape(1, -1),
      hosc2_g.reshape(1, -1), hosc2_be.reshape(1, -1), eh_w.T,
      eh_b.reshape(1, -1), nh_w.T, nh_b.reshape(1, -1),
      X_n, X_e, A_tilde, L1_tilde, B1)

    return out[:, 34], out[:, 35], out[:, :34]
